# baseline (speedup 1.0000x reference)
import math
from functools import partial

import jax
import jax.numpy as jnp
from jax.experimental import pallas as pl
from jax.experimental.pallas import tpu as pltpu

_RATIO = 0.6


def _pad_size(n: int) -> int:
    return max(512, ((n + 255) // 256) * 256)


def _zmul_kernel(h_ref, w_ref, d_ref, o_ref):
    o_ref[...] = d_ref[...] * jnp.dot(
        h_ref[...], w_ref[...], preferred_element_type=jnp.float32)


def _agg_kernel(at_ref, z_ref, zi_ref, d_ref, b_ref, o_ref, *,
                nsteps, relu, k_logical, bm):
    j = pl.program_id(1)

    @pl.when(j == 0)
    def _init():
        o_ref[...] = zi_ref[...]

    o_ref[...] += jnp.dot(at_ref[...], z_ref[...],
                          preferred_element_type=jnp.float32)

    @pl.when(j == nsteps - 1)
    def _fin():
        i = pl.program_id(0)
        acc = o_ref[...] * d_ref[...] + b_ref[...]
        if relu:
            acc = jnp.maximum(acc, 0.0)
        rows = i * bm + jax.lax.broadcasted_iota(jnp.int32, acc.shape, 0)
        o_ref[...] = jnp.where(rows < k_logical, acc, 0.0)


def _sqslice_kernel(gr_ref, gc_ref, o_ref, *, nsteps):
    kb = pl.program_id(2)

    @pl.when(kb == 0)
    def _init():
        o_ref[...] = jnp.zeros_like(o_ref)

    o_ref[...] += jax.lax.dot_general(
        gr_ref[...], gc_ref[...], (((1,), (1,)), ((), ())),
        preferred_element_type=jnp.float32)

    @pl.when(kb == nsteps - 1)
    def _fin():
        i = pl.program_id(0)
        j = pl.program_id(1)

        @pl.when(i == j)
        def _diag():
            r = jax.lax.broadcasted_iota(jnp.int32, o_ref.shape, 0)
            c = jax.lax.broadcasted_iota(jnp.int32, o_ref.shape, 1)
            o_ref[...] = jnp.where(r == c, 0.0, o_ref[...])


def _block_of(m, cands):
    for c in cands:
        if m % c == 0:
            return c
    return 128


def _gcn(At, h, d, W, b, *, k_logical, relu):
    m = At.shape[0]
    kin, nout = W.shape
    d2 = d[:, None]
    bmz = _block_of(m, (512, 256))
    z = pl.pallas_call(
        _zmul_kernel,
        grid=(m // bmz,),
        in_specs=[pl.BlockSpec((bmz, kin), lambda i: (i, 0)),
                  pl.BlockSpec((kin, nout), lambda i: (0, 0)),
                  pl.BlockSpec((bmz, 1), lambda i: (i, 0))],
        out_specs=pl.BlockSpec((bmz, nout), lambda i: (i, 0)),
        out_shape=jax.ShapeDtypeStruct((m, nout), jnp.float32),
    )(h, W, d2)

    bm = _block_of(m, (256,))
    bk = _block_of(m, (512, 256))
    gi, gj = m // bm, m // bk
    out = pl.pallas_call(
        partial(_agg_kernel, nsteps=gj, relu=relu,
                k_logical=k_logical, bm=bm),
        grid=(gi, gj),
        in_specs=[pl.BlockSpec((bm, bk), lambda i, j: (i, j)),
                  pl.BlockSpec((bk, nout), lambda i, j: (j, 0)),
                  pl.BlockSpec((bm, nout), lambda i, j: (i, 0)),
                  pl.BlockSpec((bm, 1), lambda i, j: (i, 0)),
                  pl.BlockSpec((1, nout), lambda i, j: (0, 0))],
        out_specs=pl.BlockSpec((bm, nout), lambda i, j: (i, 0)),
        out_shape=jax.ShapeDtypeStruct((m, nout), jnp.float32),
        compiler_params=pltpu.CompilerParams(
            dimension_semantics=("parallel", "arbitrary")),
    )(At, z, z, d2, b[None, :])
    return out


def _sqslice(Gr, Gc):
    m, K = Gr.shape
    bm = _block_of(m, (512, 256))
    bkk = _block_of(K, (1024, 512, 256))
    gi, gk = m // bm, K // bkk
    return pl.pallas_call(
        partial(_sqslice_kernel, nsteps=gk),
        grid=(gi, gi, gk),
        in_specs=[pl.BlockSpec((bm, bkk), lambda i, j, kb: (i, kb)),
                  pl.BlockSpec((bm, bkk), lambda i, j, kb: (j, kb))],
        out_specs=pl.BlockSpec((bm, bm), lambda i, j, kb: (i, j)),
        out_shape=jax.ShapeDtypeStruct((m, m), jnp.float32),
        compiler_params=pltpu.CompilerParams(
            dimension_semantics=("parallel", "parallel", "arbitrary")),
    )(Gr, Gc)


def kernel(x, edge_index, w_in, b_in, w_down, b_down, w_pool, w_up, b_up,
           w_out, b_out):
    N = x.shape[0]
    depth = w_down.shape[0]

    ns = [N]
    for _ in range(depth):
        ns.append(int(math.ceil(_RATIO * ns[-1])))
    pads = [N if i == 0 and N % 256 == 0 else _pad_size(n)
            for i, n in enumerate(ns)]

    r, c = edge_index[0], edge_index[1]
    m0 = pads[0]
    At = jnp.zeros((m0, m0), jnp.float32).at[c, r].add(1.0)
    A = At.T
    deg = jnp.zeros((m0,), jnp.float32).at[c].add(1.0) + 1.0
    d = jax.lax.rsqrt(deg)
    if m0 != N:
        x = jnp.pad(x, ((0, m0 - N), (0, 0)))

    h = _gcn(At, x, d, w_in, b_in, k_logical=N, relu=True)
    xs = [h]
    Ats = [At]
    ds = [d]
    perms = []

    for i in range(depth):
        n_cur, n_next = ns[i], ns[i + 1]
        m_cur, m_next = pads[i], pads[i + 1]
        w = w_pool[i]
        score_raw = jnp.tanh((h @ w) / jnp.linalg.norm(w))
        score = jnp.where(jnp.arange(m_cur) < n_cur, score_raw, -jnp.inf)
        perm = jnp.argsort(-score)[:n_next]
        perm_pad = jnp.concatenate(
            [perm, jnp.full((m_next - n_next,), n_cur, perm.dtype)])
        sel = jnp.arange(m_next) < n_next
        ones_sel = jnp.where(sel, 1.0, 0.0)

        xp = h[perm_pad] * score_raw[perm_pad][:, None]
        xp = jnp.where(sel[:, None], xp, 0.0)

        rows_idx = jnp.arange(m_next)
        Gr = jnp.where(sel[:, None], At[perm_pad], 0.0)
        Gr = Gr.at[rows_idx, perm_pad].add(ones_sel)
        Gc = jnp.where(sel[:, None], A[perm_pad], 0.0)
        Gc = Gc.at[rows_idx, perm_pad].add(ones_sel)

        At = _sqslice(Gr, Gc)
        A = At.T
        deg = jnp.sum(At, axis=1) + 1.0
        d = jax.lax.rsqrt(deg)

        h = _gcn(At, xp, d, w_down[i], b_down[i],
                 k_logical=n_next, relu=True)
        if i < depth - 1:
            xs.append(h)
            Ats.append(At)
            ds.append(d)
        perms.append(perm_pad)

    for i in range(depth):
        j = depth - 1 - i
        res = xs[j]
        up = jnp.zeros_like(res).at[perms[j]].set(h)
        h = res + up
        if i < depth - 1:
            h = _gcn(Ats[j], h, ds[j], w_up[i], b_up[i],
                     k_logical=ns[j], relu=True)
        else:
            h = _gcn(Ats[0], h, ds[0], w_out, b_out,
                     k_logical=N, relu=False)
    return h[:N]
